# initial kernel scaffold (unmeasured)
import jax
import jax.numpy as jnp
from jax import lax
from jax.experimental import pallas as pl
from jax.experimental.pallas import tpu as pltpu

N_DEV = 16
N_ROW = 4096
N_COL = 2048
CHUNK = N_ROW // N_DEV


def _allreduce_body(x_ref, out_ref, rs_bufs, rs_send, rs_recv, ag_send, ag_recv):
    my = lax.axis_index("i")
    left = (my + N_DEV - 1) % N_DEV
    right = (my + 1) % N_DEV

    barrier = pltpu.get_barrier_semaphore()
    for nbr in (left, right):
        pl.semaphore_signal(
            barrier, inc=1, device_id=(nbr,), device_id_type=pl.DeviceIdType.MESH
        )
    pl.semaphore_wait(barrier, 2)

    for s in range(N_DEV - 1):
        send_c = (my - s) % N_DEV
        recv_c = (my - s - 1) % N_DEV
        src = x_ref.at[send_c] if s == 0 else out_ref.at[send_c]
        rdma = pltpu.make_async_remote_copy(
            src_ref=src,
            dst_ref=rs_bufs.at[s],
            send_sem=rs_send.at[s],
            recv_sem=rs_recv.at[s],
            device_id=(right,),
            device_id_type=pl.DeviceIdType.MESH,
        )
        rdma.start()
        rdma.wait()
        out_ref[recv_c] = rs_bufs[s] + x_ref[recv_c]


    for h in range(N_DEV - 1):
        c = (my + 1 - h) % N_DEV
        rdma = pltpu.make_async_remote_copy(
            src_ref=out_ref.at[c],
            dst_ref=out_ref.at[c],
            send_sem=ag_send.at[h],
            recv_sem=ag_recv.at[h],
            device_id=(right,),
            device_id_type=pl.DeviceIdType.MESH,
        )
        rdma.start()
        rdma.wait()


def kernel(x, w_mat):
    partial = jnp.dot(x, w_mat, preferred_element_type=jnp.float32)
    partial = partial.astype(jnp.bfloat16).reshape(N_DEV, CHUNK, N_COL)

    reduced = pl.pallas_call(
        _allreduce_body,
        out_shape=jax.ShapeDtypeStruct((N_DEV, CHUNK, N_COL), jnp.bfloat16),
        in_specs=[pl.BlockSpec(memory_space=pltpu.VMEM)],
        out_specs=pl.BlockSpec(memory_space=pltpu.VMEM),
        scratch_shapes=[
            pltpu.VMEM((N_DEV - 1, CHUNK, N_COL), jnp.bfloat16),
            pltpu.SemaphoreType.DMA((N_DEV - 1,)),
            pltpu.SemaphoreType.DMA((N_DEV - 1,)),
            pltpu.SemaphoreType.DMA((N_DEV - 1,)),
            pltpu.SemaphoreType.DMA((N_DEV - 1,)),
        ],
        compiler_params=pltpu.CompilerParams(collective_id=0),
    )(partial)

    y = reduced.reshape(N_ROW, N_COL).astype(jnp.float32)
    y = jnp.maximum(y, 0.0)
    scale = jnp.max(y) / 448.0
    q = (y / scale).astype(jnp.float8_e4m3fn)
    return q.astype(jnp.float32) * scale


# baseline (device time: 797850 ns/iter reference)
import jax
import jax.numpy as jnp
from jax import lax
from jax.experimental import pallas as pl
from jax.experimental.pallas import tpu as pltpu

N_DEV = 16
N_ROW = 4096
N_COL = 2048
CHUNK = N_ROW // N_DEV


def _allreduce_body(
    x_hbm, out_hbm, rs_land, x_stage, rs_send, rs_recv, ag_send, ag_recv, loc_sems
):
    my = lax.axis_index("i")
    left = (my + N_DEV - 1) % N_DEV
    right = (my + 1) % N_DEV

    barrier = pltpu.get_barrier_semaphore()
    for nbr in (left, right):
        pl.semaphore_signal(
            barrier, inc=1, device_id=(nbr,), device_id_type=pl.DeviceIdType.MESH
        )
    pl.semaphore_wait(barrier, 2)

    for s in range(N_DEV - 1):
        send_c = (my - s) % N_DEV
        recv_c = (my - s - 1) % N_DEV
        pre = pltpu.make_async_copy(
            x_hbm.at[recv_c], x_stage.at[s % 2], loc_sems.at[s % 2]
        )
        pre.start()
        src = x_hbm.at[send_c] if s == 0 else rs_land.at[s - 1]
        rdma = pltpu.make_async_remote_copy(
            src_ref=src,
            dst_ref=rs_land.at[s],
            send_sem=rs_send.at[s],
            recv_sem=rs_recv.at[s],
            device_id=(right,),
            device_id_type=pl.DeviceIdType.MESH,
        )
        rdma.start()
        rdma.wait()
        pre.wait()
        rs_land[s] = rs_land[s] + x_stage[s % 2]

    own = (my + 1) % N_DEV
    cp = pltpu.make_async_copy(rs_land.at[N_DEV - 2], out_hbm.at[own], loc_sems.at[0])
    cp.start()
    cp.wait()

    for h in range(N_DEV - 1):
        c = (my + 1 - h) % N_DEV
        src = rs_land.at[N_DEV - 2] if h == 0 else out_hbm.at[c]
        rdma = pltpu.make_async_remote_copy(
            src_ref=src,
            dst_ref=out_hbm.at[c],
            send_sem=ag_send.at[h],
            recv_sem=ag_recv.at[h],
            device_id=(right,),
            device_id_type=pl.DeviceIdType.MESH,
        )
        rdma.start()
        rdma.wait()


def kernel(x, w_mat):
    partial = jnp.dot(x, w_mat, preferred_element_type=jnp.float32)
    partial = partial.reshape(N_DEV, CHUNK, N_COL)

    reduced = pl.pallas_call(
        _allreduce_body,
        out_shape=jax.ShapeDtypeStruct((N_DEV, CHUNK, N_COL), jnp.float32),
        in_specs=[pl.BlockSpec(memory_space=pl.ANY)],
        out_specs=pl.BlockSpec(memory_space=pl.ANY),
        scratch_shapes=[
            pltpu.VMEM((N_DEV - 1, CHUNK, N_COL), jnp.float32),
            pltpu.VMEM((2, CHUNK, N_COL), jnp.float32),
            pltpu.SemaphoreType.DMA((N_DEV - 1,)),
            pltpu.SemaphoreType.DMA((N_DEV - 1,)),
            pltpu.SemaphoreType.DMA((N_DEV - 1,)),
            pltpu.SemaphoreType.DMA((N_DEV - 1,)),
            pltpu.SemaphoreType.DMA((2,)),
        ],
        compiler_params=pltpu.CompilerParams(
            collective_id=0, vmem_limit_bytes=50 * 1024 * 1024
        ),
    )(partial)

    y = reduced.reshape(N_ROW, N_COL)
    y = jnp.maximum(y, 0.0)
    scale = jnp.max(y) / 448.0
    q = (y / scale).astype(jnp.float8_e4m3fn)
    q = lax.optimization_barrier(q)
    return q.astype(jnp.float32) * scale


# device time: 503404 ns/iter; 1.5849x vs baseline; 1.5849x over previous
import jax
import jax.numpy as jnp
from jax import lax
from jax.experimental import pallas as pl
from jax.experimental.pallas import tpu as pltpu

N_DEV = 16
N_ROW = 4096
N_COL = 2048
HALF = N_COL // 2
CHUNK = N_ROW // N_DEV


def _allreduce_body(
    x_hbm,
    out_hbm,
    rs_land_r,
    rs_land_l,
    x_stage,
    rsr_send,
    rsr_recv,
    rsl_send,
    rsl_recv,
    agr_send,
    agr_recv,
    agl_send,
    agl_recv,
    loc_sems,
):
    my = lax.axis_index("i")
    left = (my + N_DEV - 1) % N_DEV
    right = (my + 1) % N_DEV

    barrier = pltpu.get_barrier_semaphore()
    for nbr in (left, right):
        pl.semaphore_signal(
            barrier, inc=1, device_id=(nbr,), device_id_type=pl.DeviceIdType.MESH
        )
    pl.semaphore_wait(barrier, 2)

    for s in range(N_DEV - 1):
        send_r = (my - s) % N_DEV
        recv_r = (my - s - 1) % N_DEV
        send_l = (my + s) % N_DEV
        recv_l = (my + s + 1) % N_DEV

        pre_r = pltpu.make_async_copy(
            x_hbm.at[recv_r, :, 0:HALF], x_stage.at[s % 2, 0], loc_sems.at[s % 2, 0]
        )
        pre_l = pltpu.make_async_copy(
            x_hbm.at[recv_l, :, HALF:N_COL],
            x_stage.at[s % 2, 1],
            loc_sems.at[s % 2, 1],
        )
        pre_r.start()
        pre_l.start()

        src_r = x_hbm.at[send_r, :, 0:HALF] if s == 0 else rs_land_r.at[s - 1]
        rdma_r = pltpu.make_async_remote_copy(
            src_ref=src_r,
            dst_ref=rs_land_r.at[s],
            send_sem=rsr_send.at[s],
            recv_sem=rsr_recv.at[s],
            device_id=(right,),
            device_id_type=pl.DeviceIdType.MESH,
        )
        src_l = x_hbm.at[send_l, :, HALF:N_COL] if s == 0 else rs_land_l.at[s - 1]
        rdma_l = pltpu.make_async_remote_copy(
            src_ref=src_l,
            dst_ref=rs_land_l.at[s],
            send_sem=rsl_send.at[s],
            recv_sem=rsl_recv.at[s],
            device_id=(left,),
            device_id_type=pl.DeviceIdType.MESH,
        )
        rdma_r.start()
        rdma_l.start()
        rdma_r.wait()
        rdma_l.wait()
        pre_r.wait()
        pre_l.wait()
        rs_land_r[s] = rs_land_r[s] + x_stage[s % 2, 0]
        rs_land_l[s] = rs_land_l[s] + x_stage[s % 2, 1]

    own_r = (my + 1) % N_DEV
    own_l = (my + N_DEV - 1) % N_DEV
    cp_r = pltpu.make_async_copy(
        rs_land_r.at[N_DEV - 2], out_hbm.at[own_r, :, 0:HALF], loc_sems.at[0, 0]
    )
    cp_l = pltpu.make_async_copy(
        rs_land_l.at[N_DEV - 2], out_hbm.at[own_l, :, HALF:N_COL], loc_sems.at[0, 1]
    )
    cp_r.start()
    cp_l.start()
    cp_r.wait()
    cp_l.wait()

    for h in range(N_DEV - 1):
        c_r = (my + 1 - h) % N_DEV
        c_l = (my + N_DEV - 1 + h) % N_DEV
        src_r = rs_land_r.at[N_DEV - 2] if h == 0 else out_hbm.at[c_r, :, 0:HALF]
        rdma_r = pltpu.make_async_remote_copy(
            src_ref=src_r,
            dst_ref=out_hbm.at[c_r, :, 0:HALF],
            send_sem=agr_send.at[h],
            recv_sem=agr_recv.at[h],
            device_id=(right,),
            device_id_type=pl.DeviceIdType.MESH,
        )
        src_l = (
            rs_land_l.at[N_DEV - 2] if h == 0 else out_hbm.at[c_l, :, HALF:N_COL]
        )
        rdma_l = pltpu.make_async_remote_copy(
            src_ref=src_l,
            dst_ref=out_hbm.at[c_l, :, HALF:N_COL],
            send_sem=agl_send.at[h],
            recv_sem=agl_recv.at[h],
            device_id=(left,),
            device_id_type=pl.DeviceIdType.MESH,
        )
        rdma_r.start()
        rdma_l.start()
        rdma_r.wait()
        rdma_l.wait()


def kernel(x, w_mat):
    partial = jnp.dot(x, w_mat, preferred_element_type=jnp.float32)
    partial = partial.reshape(N_DEV, CHUNK, N_COL)

    n_hop = N_DEV - 1
    reduced = pl.pallas_call(
        _allreduce_body,
        out_shape=jax.ShapeDtypeStruct((N_DEV, CHUNK, N_COL), jnp.float32),
        in_specs=[pl.BlockSpec(memory_space=pl.ANY)],
        out_specs=pl.BlockSpec(memory_space=pl.ANY),
        scratch_shapes=[
            pltpu.VMEM((n_hop, CHUNK, HALF), jnp.float32),
            pltpu.VMEM((n_hop, CHUNK, HALF), jnp.float32),
            pltpu.VMEM((2, 2, CHUNK, HALF), jnp.float32),
            pltpu.SemaphoreType.DMA((n_hop,)),
            pltpu.SemaphoreType.DMA((n_hop,)),
            pltpu.SemaphoreType.DMA((n_hop,)),
            pltpu.SemaphoreType.DMA((n_hop,)),
            pltpu.SemaphoreType.DMA((n_hop,)),
            pltpu.SemaphoreType.DMA((n_hop,)),
            pltpu.SemaphoreType.DMA((n_hop,)),
            pltpu.SemaphoreType.DMA((n_hop,)),
            pltpu.SemaphoreType.DMA((2, 2)),
        ],
        compiler_params=pltpu.CompilerParams(
            collective_id=0, vmem_limit_bytes=50 * 1024 * 1024
        ),
    )(partial)

    y = reduced.reshape(N_ROW, N_COL)
    y = jnp.maximum(y, 0.0)
    scale = jnp.max(y) / 448.0
    q = (y / scale).astype(jnp.float8_e4m3fn)
    q = lax.optimization_barrier(q)
    return q.astype(jnp.float32) * scale


# device time: 356762 ns/iter; 2.2364x vs baseline; 1.4110x over previous
import jax
import jax.numpy as jnp
from jax import lax
from jax.experimental import pallas as pl
from jax.experimental.pallas import tpu as pltpu

N_DEV = 16
N_ROW = 4096
N_COL = 2048
HALF = N_COL // 2
CHUNK = N_ROW // N_DEV
F8 = jnp.float8_e4m3fn


def _allreduce_body(
    x_hbm,
    q_hbm,
    amax_ref,
    rs_land_r,
    rs_land_l,
    x_stage,
    q_stage,
    amax_buf,
    rsr_send,
    rsr_recv,
    rsl_send,
    rsl_recv,
    agr_send,
    agr_recv,
    agl_send,
    agl_recv,
    am_send,
    am_recv,
    loc_sems,
):
    my = lax.axis_index("i")
    left = (my + N_DEV - 1) % N_DEV
    right = (my + 1) % N_DEV

    barrier = pltpu.get_barrier_semaphore()
    for nbr in (left, right):
        pl.semaphore_signal(
            barrier, inc=1, device_id=(nbr,), device_id_type=pl.DeviceIdType.MESH
        )
    pl.semaphore_wait(barrier, 2)

    for s in range(N_DEV - 1):
        send_r = (my - s) % N_DEV
        recv_r = (my - s - 1) % N_DEV
        send_l = (my + s) % N_DEV
        recv_l = (my + s + 1) % N_DEV

        pre_r = pltpu.make_async_copy(
            x_hbm.at[recv_r, :, 0:HALF], x_stage.at[s % 2, 0], loc_sems.at[s % 2, 0]
        )
        pre_l = pltpu.make_async_copy(
            x_hbm.at[recv_l, :, HALF:N_COL],
            x_stage.at[s % 2, 1],
            loc_sems.at[s % 2, 1],
        )
        pre_r.start()
        pre_l.start()

        src_r = x_hbm.at[send_r, :, 0:HALF] if s == 0 else rs_land_r.at[s - 1]
        rdma_r = pltpu.make_async_remote_copy(
            src_ref=src_r,
            dst_ref=rs_land_r.at[s],
            send_sem=rsr_send.at[s],
            recv_sem=rsr_recv.at[s],
            device_id=(right,),
            device_id_type=pl.DeviceIdType.MESH,
        )
        src_l = x_hbm.at[send_l, :, HALF:N_COL] if s == 0 else rs_land_l.at[s - 1]
        rdma_l = pltpu.make_async_remote_copy(
            src_ref=src_l,
            dst_ref=rs_land_l.at[s],
            send_sem=rsl_send.at[s],
            recv_sem=rsl_recv.at[s],
            device_id=(left,),
            device_id_type=pl.DeviceIdType.MESH,
        )
        rdma_r.start()
        rdma_l.start()
        rdma_r.wait()
        rdma_l.wait()
        pre_r.wait()
        pre_l.wait()
        if s < N_DEV - 2:
            rs_land_r[s] = rs_land_r[s] + x_stage[s % 2, 0]
            rs_land_l[s] = rs_land_l[s] + x_stage[s % 2, 1]
        else:
            rs_land_r[s] = jnp.maximum(rs_land_r[s] + x_stage[s % 2, 0], 0.0)
            rs_land_l[s] = jnp.maximum(rs_land_l[s] + x_stage[s % 2, 1], 0.0)

    own_r = (my + 1) % N_DEV
    own_l = (my + N_DEV - 1) % N_DEV

    m_loc = jnp.maximum(jnp.max(rs_land_r[N_DEV - 2]), jnp.max(rs_land_l[N_DEV - 2]))
    amax_buf[pl.ds(my, 1), :] = jnp.full((1, 128), m_loc, jnp.float32)
    def _am_rdma(d):
        return pltpu.make_async_remote_copy(
            src_ref=amax_buf.at[pl.ds(my, 1)],
            dst_ref=amax_buf.at[pl.ds(my, 1)],
            send_sem=am_send.at[d],
            recv_sem=am_recv.at[my],
            device_id=(d,),
            device_id_type=pl.DeviceIdType.MESH,
        )

    for d in range(N_DEV):
        @pl.when(my != d)
        def _():
            _am_rdma(d).start()
    for d in range(N_DEV):
        @pl.when(my != d)
        def _():
            _am_rdma(d).wait_send()
    for j in range(N_DEV):
        @pl.when(my != j)
        def _():
            recv = pltpu.make_async_remote_copy(
                src_ref=amax_buf.at[pl.ds(j, 1)],
                dst_ref=amax_buf.at[pl.ds(j, 1)],
                send_sem=am_send.at[j],
                recv_sem=am_recv.at[j],
                device_id=(j,),
                device_id_type=pl.DeviceIdType.MESH,
            )
            recv.wait_recv()

    g_amax = jnp.max(amax_buf[...])
    amax_ref[0, 0] = g_amax
    inv = 448.0 / g_amax

    q_stage[0] = (rs_land_r[N_DEV - 2] * inv).astype(F8)
    q_stage[1] = (rs_land_l[N_DEV - 2] * inv).astype(F8)
    cp_r = pltpu.make_async_copy(
        q_stage.at[0], q_hbm.at[own_r, :, 0:HALF], loc_sems.at[0, 0]
    )
    cp_l = pltpu.make_async_copy(
        q_stage.at[1], q_hbm.at[own_l, :, HALF:N_COL], loc_sems.at[0, 1]
    )
    cp_r.start()
    cp_l.start()
    cp_r.wait()
    cp_l.wait()

    for h in range(N_DEV - 1):
        c_r = (my + 1 - h) % N_DEV
        c_l = (my + N_DEV - 1 + h) % N_DEV
        src_r = q_stage.at[0] if h == 0 else q_hbm.at[c_r, :, 0:HALF]
        rdma_r = pltpu.make_async_remote_copy(
            src_ref=src_r,
            dst_ref=q_hbm.at[c_r, :, 0:HALF],
            send_sem=agr_send.at[h],
            recv_sem=agr_recv.at[h],
            device_id=(right,),
            device_id_type=pl.DeviceIdType.MESH,
        )
        src_l = q_stage.at[1] if h == 0 else q_hbm.at[c_l, :, HALF:N_COL]
        rdma_l = pltpu.make_async_remote_copy(
            src_ref=src_l,
            dst_ref=q_hbm.at[c_l, :, HALF:N_COL],
            send_sem=agl_send.at[h],
            recv_sem=agl_recv.at[h],
            device_id=(left,),
            device_id_type=pl.DeviceIdType.MESH,
        )
        rdma_r.start()
        rdma_l.start()
        rdma_r.wait()
        rdma_l.wait()


def kernel(x, w_mat):
    partial = jnp.dot(x, w_mat, preferred_element_type=jnp.float32)
    partial = partial.reshape(N_DEV, CHUNK, N_COL)

    n_hop = N_DEV - 1
    q, amax = pl.pallas_call(
        _allreduce_body,
        out_shape=(
            jax.ShapeDtypeStruct((N_DEV, CHUNK, N_COL), F8),
            jax.ShapeDtypeStruct((1, 1), jnp.float32),
        ),
        in_specs=[pl.BlockSpec(memory_space=pl.ANY)],
        out_specs=(
            pl.BlockSpec(memory_space=pl.ANY),
            pl.BlockSpec(memory_space=pltpu.SMEM),
        ),
        scratch_shapes=[
            pltpu.VMEM((n_hop, CHUNK, HALF), jnp.float32),
            pltpu.VMEM((n_hop, CHUNK, HALF), jnp.float32),
            pltpu.VMEM((2, 2, CHUNK, HALF), jnp.float32),
            pltpu.VMEM((2, CHUNK, HALF), F8),
            pltpu.VMEM((N_DEV, 128), jnp.float32),
            pltpu.SemaphoreType.DMA((n_hop,)),
            pltpu.SemaphoreType.DMA((n_hop,)),
            pltpu.SemaphoreType.DMA((n_hop,)),
            pltpu.SemaphoreType.DMA((n_hop,)),
            pltpu.SemaphoreType.DMA((n_hop,)),
            pltpu.SemaphoreType.DMA((n_hop,)),
            pltpu.SemaphoreType.DMA((n_hop,)),
            pltpu.SemaphoreType.DMA((n_hop,)),
            pltpu.SemaphoreType.DMA((N_DEV,)),
            pltpu.SemaphoreType.DMA((N_DEV,)),
            pltpu.SemaphoreType.DMA((2, 2)),
        ],
        compiler_params=pltpu.CompilerParams(
            collective_id=0, vmem_limit_bytes=50 * 1024 * 1024
        ),
    )(partial)

    scale = amax[0, 0] / 448.0
    return q.reshape(N_ROW, N_COL).astype(jnp.float32) * scale


# device time: 345387 ns/iter; 2.3100x vs baseline; 1.0329x over previous
import jax
import jax.numpy as jnp
from jax import lax
from jax.experimental import pallas as pl
from jax.experimental.pallas import tpu as pltpu

N_DEV = 16
N_ROW = 4096
N_COL = 2048
HALF = N_COL // 2
CHUNK = N_ROW // N_DEV
F8 = jnp.float8_e4m3fn


def _allreduce_body(
    x_hbm,
    q_hbm,
    amax_ref,
    rs_land_r,
    rs_land_l,
    x_stage,
    q_stage,
    amax_buf,
    rsr_send,
    rsr_recv,
    rsl_send,
    rsl_recv,
    ag_send,
    ag_recv,
    am_send,
    am_recv,
    loc_sems,
):
    my = lax.axis_index("i")
    left = (my + N_DEV - 1) % N_DEV
    right = (my + 1) % N_DEV

    barrier = pltpu.get_barrier_semaphore()
    for nbr in (left, right):
        pl.semaphore_signal(
            barrier, inc=1, device_id=(nbr,), device_id_type=pl.DeviceIdType.MESH
        )
    pl.semaphore_wait(barrier, 2)

    for s in range(N_DEV - 1):
        send_r = (my - s) % N_DEV
        recv_r = (my - s - 1) % N_DEV
        send_l = (my + s) % N_DEV
        recv_l = (my + s + 1) % N_DEV

        pre_r = pltpu.make_async_copy(
            x_hbm.at[recv_r, :, 0:HALF], x_stage.at[s % 2, 0], loc_sems.at[s % 2, 0]
        )
        pre_l = pltpu.make_async_copy(
            x_hbm.at[recv_l, :, HALF:N_COL],
            x_stage.at[s % 2, 1],
            loc_sems.at[s % 2, 1],
        )
        pre_r.start()
        pre_l.start()

        src_r = x_hbm.at[send_r, :, 0:HALF] if s == 0 else rs_land_r.at[s - 1]
        rdma_r = pltpu.make_async_remote_copy(
            src_ref=src_r,
            dst_ref=rs_land_r.at[s],
            send_sem=rsr_send.at[s],
            recv_sem=rsr_recv.at[s],
            device_id=(right,),
            device_id_type=pl.DeviceIdType.MESH,
        )
        src_l = x_hbm.at[send_l, :, HALF:N_COL] if s == 0 else rs_land_l.at[s - 1]
        rdma_l = pltpu.make_async_remote_copy(
            src_ref=src_l,
            dst_ref=rs_land_l.at[s],
            send_sem=rsl_send.at[s],
            recv_sem=rsl_recv.at[s],
            device_id=(left,),
            device_id_type=pl.DeviceIdType.MESH,
        )
        rdma_r.start()
        rdma_l.start()
        rdma_r.wait()
        rdma_l.wait()
        pre_r.wait()
        pre_l.wait()
        if s < N_DEV - 2:
            rs_land_r[s] = rs_land_r[s] + x_stage[s % 2, 0]
            rs_land_l[s] = rs_land_l[s] + x_stage[s % 2, 1]
        else:
            rs_land_r[s] = jnp.maximum(rs_land_r[s] + x_stage[s % 2, 0], 0.0)
            rs_land_l[s] = jnp.maximum(rs_land_l[s] + x_stage[s % 2, 1], 0.0)

    own_r = (my + 1) % N_DEV
    own_l = (my + N_DEV - 1) % N_DEV

    m_loc = jnp.maximum(jnp.max(rs_land_r[N_DEV - 2]), jnp.max(rs_land_l[N_DEV - 2]))
    amax_buf[pl.ds(my, 1), :] = jnp.full((1, 128), m_loc, jnp.float32)
    def _am_rdma(d):
        return pltpu.make_async_remote_copy(
            src_ref=amax_buf.at[pl.ds(my, 1)],
            dst_ref=amax_buf.at[pl.ds(my, 1)],
            send_sem=am_send.at[d],
            recv_sem=am_recv.at[my],
            device_id=(d,),
            device_id_type=pl.DeviceIdType.MESH,
        )

    for d in range(N_DEV):
        @pl.when(my != d)
        def _():
            _am_rdma(d).start()
    for d in range(N_DEV):
        @pl.when(my != d)
        def _():
            _am_rdma(d).wait_send()
    for j in range(N_DEV):
        @pl.when(my != j)
        def _():
            recv = pltpu.make_async_remote_copy(
                src_ref=amax_buf.at[pl.ds(j, 1)],
                dst_ref=amax_buf.at[pl.ds(j, 1)],
                send_sem=am_send.at[j],
                recv_sem=am_recv.at[j],
                device_id=(j,),
                device_id_type=pl.DeviceIdType.MESH,
            )
            recv.wait_recv()

    g_amax = jnp.max(amax_buf[...])
    amax_ref[0, 0] = g_amax
    inv = 448.0 / g_amax

    q_stage[0] = (rs_land_r[N_DEV - 2] * inv).astype(F8)
    q_stage[1] = (rs_land_l[N_DEV - 2] * inv).astype(F8)
    cp_r = pltpu.make_async_copy(
        q_stage.at[0], q_hbm.at[own_r, :, 0:HALF], loc_sems.at[0, 0]
    )
    cp_l = pltpu.make_async_copy(
        q_stage.at[1], q_hbm.at[own_l, :, HALF:N_COL], loc_sems.at[0, 1]
    )
    cp_r.start()
    cp_l.start()
    cp_r.wait()
    cp_l.wait()

    def _ag_send(d, half):
        if half == 0:
            src, dst = q_stage.at[0], q_hbm.at[own_r, :, 0:HALF]
        else:
            src, dst = q_stage.at[1], q_hbm.at[own_l, :, HALF:N_COL]
        return pltpu.make_async_remote_copy(
            src_ref=src,
            dst_ref=dst,
            send_sem=ag_send.at[d, half],
            recv_sem=ag_recv.at[my, half],
            device_id=(d,),
            device_id_type=pl.DeviceIdType.MESH,
        )

    for d in range(N_DEV):
        @pl.when(my != d)
        def _():
            _ag_send(d, 0).start()
            _ag_send(d, 1).start()
    for d in range(N_DEV):
        @pl.when(my != d)
        def _():
            _ag_send(d, 0).wait_send()
            _ag_send(d, 1).wait_send()
    for j in range(N_DEV):
        cr = (j + 1) % N_DEV
        cl = (j + N_DEV - 1) % N_DEV

        @pl.when(my != j)
        def _():
            recv_r = pltpu.make_async_remote_copy(
                src_ref=q_stage.at[0],
                dst_ref=q_hbm.at[cr, :, 0:HALF],
                send_sem=ag_send.at[j, 0],
                recv_sem=ag_recv.at[j, 0],
                device_id=(j,),
                device_id_type=pl.DeviceIdType.MESH,
            )
            recv_l = pltpu.make_async_remote_copy(
                src_ref=q_stage.at[1],
                dst_ref=q_hbm.at[cl, :, HALF:N_COL],
                send_sem=ag_send.at[j, 1],
                recv_sem=ag_recv.at[j, 1],
                device_id=(j,),
                device_id_type=pl.DeviceIdType.MESH,
            )
            recv_r.wait_recv()
            recv_l.wait_recv()


def kernel(x, w_mat):
    partial = jnp.dot(x, w_mat, preferred_element_type=jnp.float32)
    partial = partial.reshape(N_DEV, CHUNK, N_COL)

    n_hop = N_DEV - 1
    q, amax = pl.pallas_call(
        _allreduce_body,
        out_shape=(
            jax.ShapeDtypeStruct((N_DEV, CHUNK, N_COL), F8),
            jax.ShapeDtypeStruct((1, 1), jnp.float32),
        ),
        in_specs=[pl.BlockSpec(memory_space=pl.ANY)],
        out_specs=(
            pl.BlockSpec(memory_space=pl.ANY),
            pl.BlockSpec(memory_space=pltpu.SMEM),
        ),
        scratch_shapes=[
            pltpu.VMEM((n_hop, CHUNK, HALF), jnp.float32),
            pltpu.VMEM((n_hop, CHUNK, HALF), jnp.float32),
            pltpu.VMEM((2, 2, CHUNK, HALF), jnp.float32),
            pltpu.VMEM((2, CHUNK, HALF), F8),
            pltpu.VMEM((N_DEV, 128), jnp.float32),
            pltpu.SemaphoreType.DMA((n_hop,)),
            pltpu.SemaphoreType.DMA((n_hop,)),
            pltpu.SemaphoreType.DMA((n_hop,)),
            pltpu.SemaphoreType.DMA((n_hop,)),
            pltpu.SemaphoreType.DMA((N_DEV, 2)),
            pltpu.SemaphoreType.DMA((N_DEV, 2)),
            pltpu.SemaphoreType.DMA((N_DEV,)),
            pltpu.SemaphoreType.DMA((N_DEV,)),
            pltpu.SemaphoreType.DMA((2, 2)),
        ],
        compiler_params=pltpu.CompilerParams(
            collective_id=0, vmem_limit_bytes=50 * 1024 * 1024
        ),
    )(partial)

    scale = amax[0, 0] / 448.0
    return q.reshape(N_ROW, N_COL).astype(jnp.float32) * scale


# device time: 259805 ns/iter; 3.0710x vs baseline; 1.3294x over previous
import jax
import jax.numpy as jnp
from jax import lax
from jax.experimental import pallas as pl
from jax.experimental.pallas import tpu as pltpu

N_DEV = 16
N_ROW = 4096
N_COL = 2048
HALF = N_COL // 2
CHUNK = N_ROW // N_DEV
F8 = jnp.float8_e4m3fn


def _allreduce_body(
    x_hbm,
    q_hbm,
    amax_ref,
    rs_land_r,
    rs_land_l,
    sb_r,
    sb_l,
    own_f32,
    x_stage,
    q_stage,
    amax_buf,
    rsr_send,
    rsr_recv,
    rsl_send,
    rsl_recv,
    ag_send,
    ag_recv,
    am_send,
    am_recv,
    loc_sems,
):
    my = lax.axis_index("i")
    left = (my + N_DEV - 1) % N_DEV
    right = (my + 1) % N_DEV

    barrier = pltpu.get_barrier_semaphore()
    for nbr in (left, right):
        pl.semaphore_signal(
            barrier, inc=1, device_id=(nbr,), device_id_type=pl.DeviceIdType.MESH
        )
    pl.semaphore_wait(barrier, 2)

    pre0_r = pltpu.make_async_copy(
        x_hbm.at[my, :, 0:HALF], x_stage.at[0, 0], loc_sems.at[0, 0]
    )
    pre0_l = pltpu.make_async_copy(
        x_hbm.at[my, :, HALF:N_COL], x_stage.at[0, 1], loc_sems.at[0, 1]
    )
    pre0_r.start()
    pre0_l.start()
    pre0_r.wait()
    pre0_l.wait()
    sb_r[0] = x_stage[0, 0].astype(jnp.bfloat16)
    sb_l[0] = x_stage[0, 1].astype(jnp.bfloat16)

    for s in range(N_DEV - 1):
        recv_r = (my - s - 1) % N_DEV
        recv_l = (my + s + 1) % N_DEV

        rdma_r = pltpu.make_async_remote_copy(
            src_ref=sb_r.at[s],
            dst_ref=rs_land_r.at[s],
            send_sem=rsr_send.at[s],
            recv_sem=rsr_recv.at[s],
            device_id=(right,),
            device_id_type=pl.DeviceIdType.MESH,
        )
        rdma_l = pltpu.make_async_remote_copy(
            src_ref=sb_l.at[s],
            dst_ref=rs_land_l.at[s],
            send_sem=rsl_send.at[s],
            recv_sem=rsl_recv.at[s],
            device_id=(left,),
            device_id_type=pl.DeviceIdType.MESH,
        )
        rdma_r.start()
        rdma_l.start()

        slot = (s + 1) % 2
        pre_r = pltpu.make_async_copy(
            x_hbm.at[recv_r, :, 0:HALF], x_stage.at[slot, 0], loc_sems.at[slot, 0]
        )
        pre_l = pltpu.make_async_copy(
            x_hbm.at[recv_l, :, HALF:N_COL], x_stage.at[slot, 1], loc_sems.at[slot, 1]
        )
        pre_r.start()
        pre_l.start()

        rdma_r.wait()
        rdma_l.wait()
        pre_r.wait()
        pre_l.wait()
        acc_r = rs_land_r[s].astype(jnp.float32) + x_stage[slot, 0]
        acc_l = rs_land_l[s].astype(jnp.float32) + x_stage[slot, 1]
        if s < N_DEV - 2:
            sb_r[s + 1] = acc_r.astype(jnp.bfloat16)
            sb_l[s + 1] = acc_l.astype(jnp.bfloat16)
        else:
            own_f32[0] = jnp.maximum(acc_r, 0.0)
            own_f32[1] = jnp.maximum(acc_l, 0.0)

    own_r = (my + 1) % N_DEV
    own_l = (my + N_DEV - 1) % N_DEV

    m_loc = jnp.maximum(jnp.max(own_f32[0]), jnp.max(own_f32[1]))
    amax_buf[pl.ds(my, 1), :] = jnp.full((1, 128), m_loc, jnp.float32)
    def _am_rdma(d):
        return pltpu.make_async_remote_copy(
            src_ref=amax_buf.at[pl.ds(my, 1)],
            dst_ref=amax_buf.at[pl.ds(my, 1)],
            send_sem=am_send.at[d],
            recv_sem=am_recv.at[my],
            device_id=(d,),
            device_id_type=pl.DeviceIdType.MESH,
        )

    for d in range(N_DEV):
        @pl.when(my != d)
        def _():
            _am_rdma(d).start()
    for d in range(N_DEV):
        @pl.when(my != d)
        def _():
            _am_rdma(d).wait_send()
    for j in range(N_DEV):
        @pl.when(my != j)
        def _():
            recv = pltpu.make_async_remote_copy(
                src_ref=amax_buf.at[pl.ds(j, 1)],
                dst_ref=amax_buf.at[pl.ds(j, 1)],
                send_sem=am_send.at[j],
                recv_sem=am_recv.at[j],
                device_id=(j,),
                device_id_type=pl.DeviceIdType.MESH,
            )
            recv.wait_recv()

    g_amax = jnp.max(amax_buf[...])
    amax_ref[0, 0] = g_amax
    inv = 448.0 / g_amax

    q_stage[0] = (own_f32[0] * inv).astype(F8)
    q_stage[1] = (own_f32[1] * inv).astype(F8)
    cp_r = pltpu.make_async_copy(
        q_stage.at[0], q_hbm.at[own_r, :, 0:HALF], loc_sems.at[0, 0]
    )
    cp_l = pltpu.make_async_copy(
        q_stage.at[1], q_hbm.at[own_l, :, HALF:N_COL], loc_sems.at[0, 1]
    )
    cp_r.start()
    cp_l.start()
    cp_r.wait()
    cp_l.wait()

    def _ag_send(d, half):
        if half == 0:
            src, dst = q_stage.at[0], q_hbm.at[own_r, :, 0:HALF]
        else:
            src, dst = q_stage.at[1], q_hbm.at[own_l, :, HALF:N_COL]
        return pltpu.make_async_remote_copy(
            src_ref=src,
            dst_ref=dst,
            send_sem=ag_send.at[d, half],
            recv_sem=ag_recv.at[my, half],
            device_id=(d,),
            device_id_type=pl.DeviceIdType.MESH,
        )

    for d in range(N_DEV):
        @pl.when(my != d)
        def _():
            _ag_send(d, 0).start()
            _ag_send(d, 1).start()
    for d in range(N_DEV):
        @pl.when(my != d)
        def _():
            _ag_send(d, 0).wait_send()
            _ag_send(d, 1).wait_send()
    for j in range(N_DEV):
        cr = (j + 1) % N_DEV
        cl = (j + N_DEV - 1) % N_DEV

        @pl.when(my != j)
        def _():
            recv_r = pltpu.make_async_remote_copy(
                src_ref=q_stage.at[0],
                dst_ref=q_hbm.at[cr, :, 0:HALF],
                send_sem=ag_send.at[j, 0],
                recv_sem=ag_recv.at[j, 0],
                device_id=(j,),
                device_id_type=pl.DeviceIdType.MESH,
            )
            recv_l = pltpu.make_async_remote_copy(
                src_ref=q_stage.at[1],
                dst_ref=q_hbm.at[cl, :, HALF:N_COL],
                send_sem=ag_send.at[j, 1],
                recv_sem=ag_recv.at[j, 1],
                device_id=(j,),
                device_id_type=pl.DeviceIdType.MESH,
            )
            recv_r.wait_recv()
            recv_l.wait_recv()


def kernel(x, w_mat):
    partial = jnp.dot(x, w_mat, preferred_element_type=jnp.float32)
    partial = partial.reshape(N_DEV, CHUNK, N_COL)

    n_hop = N_DEV - 1
    q, amax = pl.pallas_call(
        _allreduce_body,
        out_shape=(
            jax.ShapeDtypeStruct((N_DEV, CHUNK, N_COL), F8),
            jax.ShapeDtypeStruct((1, 1), jnp.float32),
        ),
        in_specs=[pl.BlockSpec(memory_space=pl.ANY)],
        out_specs=(
            pl.BlockSpec(memory_space=pl.ANY),
            pl.BlockSpec(memory_space=pltpu.SMEM),
        ),
        scratch_shapes=[
            pltpu.VMEM((n_hop, CHUNK, HALF), jnp.bfloat16),
            pltpu.VMEM((n_hop, CHUNK, HALF), jnp.bfloat16),
            pltpu.VMEM((n_hop, CHUNK, HALF), jnp.bfloat16),
            pltpu.VMEM((n_hop, CHUNK, HALF), jnp.bfloat16),
            pltpu.VMEM((2, CHUNK, HALF), jnp.float32),
            pltpu.VMEM((2, 2, CHUNK, HALF), jnp.float32),
            pltpu.VMEM((2, CHUNK, HALF), F8),
            pltpu.VMEM((N_DEV, 128), jnp.float32),
            pltpu.SemaphoreType.DMA((n_hop,)),
            pltpu.SemaphoreType.DMA((n_hop,)),
            pltpu.SemaphoreType.DMA((n_hop,)),
            pltpu.SemaphoreType.DMA((n_hop,)),
            pltpu.SemaphoreType.DMA((N_DEV, 2)),
            pltpu.SemaphoreType.DMA((N_DEV, 2)),
            pltpu.SemaphoreType.DMA((N_DEV,)),
            pltpu.SemaphoreType.DMA((N_DEV,)),
            pltpu.SemaphoreType.DMA((2, 2)),
        ],
        compiler_params=pltpu.CompilerParams(
            collective_id=0, vmem_limit_bytes=50 * 1024 * 1024
        ),
    )(partial)

    scale = amax[0, 0] / 448.0
    return q.reshape(N_ROW, N_COL).astype(jnp.float32) * scale


# device time: 259139 ns/iter; 3.0788x vs baseline; 1.0026x over previous
import jax
import jax.numpy as jnp
from jax import lax
from jax.experimental import pallas as pl
from jax.experimental.pallas import tpu as pltpu

N_DEV = 16
N_ROW = 4096
N_COL = 2048
HALF = N_COL // 2
CHUNK = N_ROW // N_DEV
F8 = jnp.float8_e4m3fn


def _allreduce_body(
    x_hbm,
    q_hbm,
    amax_ref,
    rs_land_r,
    rs_land_l,
    sb_r,
    sb_l,
    own_f32,
    x_stage,
    q_stage,
    amax_buf,
    rsr_send,
    rsr_recv,
    rsl_send,
    rsl_recv,
    ag_send,
    ag_recv,
    am_send,
    am_recv,
    loc_sems,
):
    my = lax.axis_index("i")
    left = (my + N_DEV - 1) % N_DEV
    right = (my + 1) % N_DEV

    barrier = pltpu.get_barrier_semaphore()
    for nbr in (left, right):
        pl.semaphore_signal(
            barrier, inc=1, device_id=(nbr,), device_id_type=pl.DeviceIdType.MESH
        )
    pl.semaphore_wait(barrier, 2)

    pre0_r = pltpu.make_async_copy(
        x_hbm.at[my, :, 0:HALF], x_stage.at[0, 0], loc_sems.at[0, 0]
    )
    pre0_l = pltpu.make_async_copy(
        x_hbm.at[my, :, HALF:N_COL], x_stage.at[0, 1], loc_sems.at[0, 1]
    )
    pre0_r.start()
    pre0_l.start()
    pre0_r.wait()
    pre0_l.wait()
    sb_r[0] = x_stage[0, 0].astype(jnp.bfloat16)
    sb_l[0] = x_stage[0, 1].astype(jnp.bfloat16)

    ROWS = (slice(0, CHUNK // 2), slice(CHUNK // 2, CHUNK))
    for s in range(N_DEV - 1):
        recv_r = (my - s - 1) % N_DEV
        recv_l = (my + s + 1) % N_DEV

        rdmas = []
        for k in (0, 1):
            rdma_r = pltpu.make_async_remote_copy(
                src_ref=sb_r.at[s, ROWS[k]],
                dst_ref=rs_land_r.at[s, ROWS[k]],
                send_sem=rsr_send.at[s, k],
                recv_sem=rsr_recv.at[s, k],
                device_id=(right,),
                device_id_type=pl.DeviceIdType.MESH,
            )
            rdma_l = pltpu.make_async_remote_copy(
                src_ref=sb_l.at[s, ROWS[k]],
                dst_ref=rs_land_l.at[s, ROWS[k]],
                send_sem=rsl_send.at[s, k],
                recv_sem=rsl_recv.at[s, k],
                device_id=(left,),
                device_id_type=pl.DeviceIdType.MESH,
            )
            rdma_r.start()
            rdma_l.start()
            rdmas.append((rdma_r, rdma_l))

        slot = (s + 1) % 2
        pre_r = pltpu.make_async_copy(
            x_hbm.at[recv_r, :, 0:HALF], x_stage.at[slot, 0], loc_sems.at[slot, 0]
        )
        pre_l = pltpu.make_async_copy(
            x_hbm.at[recv_l, :, HALF:N_COL], x_stage.at[slot, 1], loc_sems.at[slot, 1]
        )
        pre_r.start()
        pre_l.start()
        pre_r.wait()
        pre_l.wait()

        for k in (0, 1):
            rdmas[k][0].wait()
            rdmas[k][1].wait()
            rk = ROWS[k]
            acc_r = rs_land_r[s, rk].astype(jnp.float32) + x_stage[slot, 0, rk]
            acc_l = rs_land_l[s, rk].astype(jnp.float32) + x_stage[slot, 1, rk]
            if s < N_DEV - 2:
                sb_r[s + 1, rk] = acc_r.astype(jnp.bfloat16)
                sb_l[s + 1, rk] = acc_l.astype(jnp.bfloat16)
            else:
                own_f32[0, rk] = jnp.maximum(acc_r, 0.0)
                own_f32[1, rk] = jnp.maximum(acc_l, 0.0)

    own_r = (my + 1) % N_DEV
    own_l = (my + N_DEV - 1) % N_DEV

    m_loc = jnp.maximum(jnp.max(own_f32[0]), jnp.max(own_f32[1]))
    amax_buf[pl.ds(my, 1), :] = jnp.full((1, 128), m_loc, jnp.float32)
    def _am_rdma(d):
        return pltpu.make_async_remote_copy(
            src_ref=amax_buf.at[pl.ds(my, 1)],
            dst_ref=amax_buf.at[pl.ds(my, 1)],
            send_sem=am_send.at[d],
            recv_sem=am_recv.at[my],
            device_id=(d,),
            device_id_type=pl.DeviceIdType.MESH,
        )

    for d in range(N_DEV):
        @pl.when(my != d)
        def _():
            _am_rdma(d).start()
    for d in range(N_DEV):
        @pl.when(my != d)
        def _():
            _am_rdma(d).wait_send()
    for j in range(N_DEV):
        @pl.when(my != j)
        def _():
            recv = pltpu.make_async_remote_copy(
                src_ref=amax_buf.at[pl.ds(j, 1)],
                dst_ref=amax_buf.at[pl.ds(j, 1)],
                send_sem=am_send.at[j],
                recv_sem=am_recv.at[j],
                device_id=(j,),
                device_id_type=pl.DeviceIdType.MESH,
            )
            recv.wait_recv()

    g_amax = jnp.max(amax_buf[...])
    amax_ref[0, 0] = g_amax
    inv = 448.0 / g_amax

    q_stage[0] = (own_f32[0] * inv).astype(F8)
    q_stage[1] = (own_f32[1] * inv).astype(F8)
    cp_r = pltpu.make_async_copy(
        q_stage.at[0], q_hbm.at[own_r, :, 0:HALF], loc_sems.at[0, 0]
    )
    cp_l = pltpu.make_async_copy(
        q_stage.at[1], q_hbm.at[own_l, :, HALF:N_COL], loc_sems.at[0, 1]
    )
    cp_r.start()
    cp_l.start()
    cp_r.wait()
    cp_l.wait()

    def _ag_send(d, half):
        if half == 0:
            src, dst = q_stage.at[0], q_hbm.at[own_r, :, 0:HALF]
        else:
            src, dst = q_stage.at[1], q_hbm.at[own_l, :, HALF:N_COL]
        return pltpu.make_async_remote_copy(
            src_ref=src,
            dst_ref=dst,
            send_sem=ag_send.at[d, half],
            recv_sem=ag_recv.at[my, half],
            device_id=(d,),
            device_id_type=pl.DeviceIdType.MESH,
        )

    for d in range(N_DEV):
        @pl.when(my != d)
        def _():
            _ag_send(d, 0).start()
            _ag_send(d, 1).start()
    for d in range(N_DEV):
        @pl.when(my != d)
        def _():
            _ag_send(d, 0).wait_send()
            _ag_send(d, 1).wait_send()
    for j in range(N_DEV):
        cr = (j + 1) % N_DEV
        cl = (j + N_DEV - 1) % N_DEV

        @pl.when(my != j)
        def _():
            recv_r = pltpu.make_async_remote_copy(
                src_ref=q_stage.at[0],
                dst_ref=q_hbm.at[cr, :, 0:HALF],
                send_sem=ag_send.at[j, 0],
                recv_sem=ag_recv.at[j, 0],
                device_id=(j,),
                device_id_type=pl.DeviceIdType.MESH,
            )
            recv_l = pltpu.make_async_remote_copy(
                src_ref=q_stage.at[1],
                dst_ref=q_hbm.at[cl, :, HALF:N_COL],
                send_sem=ag_send.at[j, 1],
                recv_sem=ag_recv.at[j, 1],
                device_id=(j,),
                device_id_type=pl.DeviceIdType.MESH,
            )
            recv_r.wait_recv()
            recv_l.wait_recv()


def kernel(x, w_mat):
    partial = jnp.dot(x, w_mat, preferred_element_type=jnp.float32)
    partial = partial.reshape(N_DEV, CHUNK, N_COL)

    n_hop = N_DEV - 1
    q, amax = pl.pallas_call(
        _allreduce_body,
        out_shape=(
            jax.ShapeDtypeStruct((N_DEV, CHUNK, N_COL), F8),
            jax.ShapeDtypeStruct((1, 1), jnp.float32),
        ),
        in_specs=[pl.BlockSpec(memory_space=pl.ANY)],
        out_specs=(
            pl.BlockSpec(memory_space=pl.ANY),
            pl.BlockSpec(memory_space=pltpu.SMEM),
        ),
        scratch_shapes=[
            pltpu.VMEM((n_hop, CHUNK, HALF), jnp.bfloat16),
            pltpu.VMEM((n_hop, CHUNK, HALF), jnp.bfloat16),
            pltpu.VMEM((n_hop, CHUNK, HALF), jnp.bfloat16),
            pltpu.VMEM((n_hop, CHUNK, HALF), jnp.bfloat16),
            pltpu.VMEM((2, CHUNK, HALF), jnp.float32),
            pltpu.VMEM((2, 2, CHUNK, HALF), jnp.float32),
            pltpu.VMEM((2, CHUNK, HALF), F8),
            pltpu.VMEM((N_DEV, 128), jnp.float32),
            pltpu.SemaphoreType.DMA((n_hop, 2)),
            pltpu.SemaphoreType.DMA((n_hop, 2)),
            pltpu.SemaphoreType.DMA((n_hop, 2)),
            pltpu.SemaphoreType.DMA((n_hop, 2)),
            pltpu.SemaphoreType.DMA((N_DEV, 2)),
            pltpu.SemaphoreType.DMA((N_DEV, 2)),
            pltpu.SemaphoreType.DMA((N_DEV,)),
            pltpu.SemaphoreType.DMA((N_DEV,)),
            pltpu.SemaphoreType.DMA((2, 2)),
        ],
        compiler_params=pltpu.CompilerParams(
            collective_id=0, vmem_limit_bytes=50 * 1024 * 1024
        ),
    )(partial)

    scale = amax[0, 0] / 448.0
    return q.reshape(N_ROW, N_COL).astype(jnp.float32) * scale


# device time: 258708 ns/iter; 3.0840x vs baseline; 1.0017x over previous
import jax
import jax.numpy as jnp
from jax import lax
from jax.experimental import pallas as pl
from jax.experimental.pallas import tpu as pltpu

N_DEV = 16
N_ROW = 4096
N_COL = 2048
HALF = N_COL // 2
CHUNK = N_ROW // N_DEV
F8 = jnp.float8_e4m3fn
N_DEQ = 4


def _body(
    x_ref,
    w_ref,
    out_hbm,
    rs_land_r,
    rs_land_l,
    sb_r,
    sb_l,
    own_f32,
    q_stage,
    ag_land,
    deq_tmp,
    amax_buf,
    rsr_send,
    rsr_recv,
    rsl_send,
    rsl_recv,
    ag_send,
    ag_recv,
    am_send,
    am_recv,
    deq_sems,
):
    my = lax.axis_index("i")
    left = (my + N_DEV - 1) % N_DEV
    right = (my + 1) % N_DEV

    barrier = pltpu.get_barrier_semaphore()
    for nbr in (left, right):
        pl.semaphore_signal(
            barrier, inc=1, device_id=(nbr,), device_id_type=pl.DeviceIdType.MESH
        )
    pl.semaphore_wait(barrier, 2)

    def partial_halves(c):
        xc = x_ref[c]
        p_r = lax.dot_general(
            xc, w_ref[:, 0:HALF], (((1,), (0,)), ((), ())),
            preferred_element_type=jnp.float32,
        )
        p_l = lax.dot_general(
            xc, w_ref[:, HALF:N_COL], (((1,), (0,)), ((), ())),
            preferred_element_type=jnp.float32,
        )
        return p_r, p_l

    p0_r, p0_l = partial_halves(my)
    sb_r[0] = p0_r.astype(jnp.bfloat16)
    sb_l[0] = p0_l.astype(jnp.bfloat16)

    for s in range(N_DEV - 1):
        recv_r = (my - s - 1) % N_DEV
        recv_l = (my + s + 1) % N_DEV

        rdma_r = pltpu.make_async_remote_copy(
            src_ref=sb_r.at[s],
            dst_ref=rs_land_r.at[s],
            send_sem=rsr_send.at[s],
            recv_sem=rsr_recv.at[s],
            device_id=(right,),
            device_id_type=pl.DeviceIdType.MESH,
        )
        rdma_l = pltpu.make_async_remote_copy(
            src_ref=sb_l.at[s],
            dst_ref=rs_land_l.at[s],
            send_sem=rsl_send.at[s],
            recv_sem=rsl_recv.at[s],
            device_id=(left,),
            device_id_type=pl.DeviceIdType.MESH,
        )
        rdma_r.start()
        rdma_l.start()

        p_r, _ = partial_halves(recv_r)
        _, p_l = partial_halves(recv_l)

        rdma_r.wait()
        rdma_l.wait()
        acc_r = rs_land_r[s].astype(jnp.float32) + p_r
        acc_l = rs_land_l[s].astype(jnp.float32) + p_l
        if s < N_DEV - 2:
            sb_r[s + 1] = acc_r.astype(jnp.bfloat16)
            sb_l[s + 1] = acc_l.astype(jnp.bfloat16)
        else:
            own_f32[0] = jnp.maximum(acc_r, 0.0)
            own_f32[1] = jnp.maximum(acc_l, 0.0)

    own_r = (my + 1) % N_DEV
    own_l = (my + N_DEV - 1) % N_DEV

    m_loc = jnp.maximum(jnp.max(own_f32[0]), jnp.max(own_f32[1]))
    amax_buf[pl.ds(my, 1), :] = jnp.full((1, 128), m_loc, jnp.float32)

    def _am_rdma(d):
        return pltpu.make_async_remote_copy(
            src_ref=amax_buf.at[pl.ds(my, 1)],
            dst_ref=amax_buf.at[pl.ds(my, 1)],
            send_sem=am_send.at[d],
            recv_sem=am_recv.at[my],
            device_id=(d,),
            device_id_type=pl.DeviceIdType.MESH,
        )

    for d in range(N_DEV):
        @pl.when(my != d)
        def _():
            _am_rdma(d).start()
    for d in range(N_DEV):
        @pl.when(my != d)
        def _():
            _am_rdma(d).wait_send()
    for j in range(N_DEV):
        @pl.when(my != j)
        def _():
            recv = pltpu.make_async_remote_copy(
                src_ref=amax_buf.at[pl.ds(j, 1)],
                dst_ref=amax_buf.at[pl.ds(j, 1)],
                send_sem=am_send.at[j],
                recv_sem=am_recv.at[j],
                device_id=(j,),
                device_id_type=pl.DeviceIdType.MESH,
            )
            recv.wait_recv()

    g_amax = jnp.max(amax_buf[...])
    inv = 448.0 / g_amax
    scale = g_amax / 448.0

    q_stage[0] = (own_f32[0] * inv).astype(F8)
    q_stage[1] = (own_f32[1] * inv).astype(F8)

    def _ag_rdma(d, half, sender):
        return pltpu.make_async_remote_copy(
            src_ref=q_stage.at[half],
            dst_ref=ag_land.at[sender, half],
            send_sem=ag_send.at[d, half],
            recv_sem=ag_recv.at[sender, half],
            device_id=(d,),
            device_id_type=pl.DeviceIdType.MESH,
        )

    for d in range(N_DEV):
        @pl.when(my != d)
        def _():
            _ag_rdma(d, 0, my).start()
            _ag_rdma(d, 1, my).start()

    dsts = []
    n = 0
    for j in range(N_DEV):
        for half in (0, 1):
            c = (j + 1) % N_DEV if half == 0 else (j + N_DEV - 1) % N_DEV
            dst = (
                out_hbm.at[c, :, 0:HALF]
                if half == 0
                else out_hbm.at[c, :, HALF:N_COL]
            )

            @pl.when(my != j)
            def _():
                _ag_rdma(j, half, j).wait_recv()

            slot = n % N_DEQ
            if n >= N_DEQ:
                pltpu.make_async_copy(
                    deq_tmp.at[slot], dsts[n - N_DEQ], deq_sems.at[slot]
                ).wait()
            src = jnp.where(
                my == j,
                q_stage[half].astype(jnp.float32),
                ag_land[j, half].astype(jnp.float32),
            )
            deq_tmp[slot] = src * scale
            pltpu.make_async_copy(deq_tmp.at[slot], dst, deq_sems.at[slot]).start()
            dsts.append(dst)
            n += 1

    for k in range(N_DEQ):
        slot = (n - N_DEQ + k) % N_DEQ
        pltpu.make_async_copy(
            deq_tmp.at[slot], dsts[n - N_DEQ + k], deq_sems.at[slot]
        ).wait()
    for d in range(N_DEV):
        @pl.when(my != d)
        def _():
            _ag_rdma(d, 0, my).wait_send()
            _ag_rdma(d, 1, my).wait_send()


def kernel(x, w_mat):
    x = x.reshape(N_DEV, CHUNK, N_DEV * CHUNK // N_DEV)

    n_hop = N_DEV - 1
    out = pl.pallas_call(
        _body,
        out_shape=jax.ShapeDtypeStruct((N_DEV, CHUNK, N_COL), jnp.float32),
        in_specs=[
            pl.BlockSpec(memory_space=pltpu.VMEM),
            pl.BlockSpec(memory_space=pltpu.VMEM),
        ],
        out_specs=pl.BlockSpec(memory_space=pl.ANY),
        scratch_shapes=[
            pltpu.VMEM((n_hop, CHUNK, HALF), jnp.bfloat16),
            pltpu.VMEM((n_hop, CHUNK, HALF), jnp.bfloat16),
            pltpu.VMEM((n_hop, CHUNK, HALF), jnp.bfloat16),
            pltpu.VMEM((n_hop, CHUNK, HALF), jnp.bfloat16),
            pltpu.VMEM((2, CHUNK, HALF), jnp.float32),
            pltpu.VMEM((2, CHUNK, HALF), F8),
            pltpu.VMEM((N_DEV, 2, CHUNK, HALF), F8),
            pltpu.VMEM((N_DEQ, CHUNK, HALF), jnp.float32),
            pltpu.VMEM((N_DEV, 128), jnp.float32),
            pltpu.SemaphoreType.DMA((n_hop,)),
            pltpu.SemaphoreType.DMA((n_hop,)),
            pltpu.SemaphoreType.DMA((n_hop,)),
            pltpu.SemaphoreType.DMA((n_hop,)),
            pltpu.SemaphoreType.DMA((N_DEV, 2)),
            pltpu.SemaphoreType.DMA((N_DEV, 2)),
            pltpu.SemaphoreType.DMA((N_DEV,)),
            pltpu.SemaphoreType.DMA((N_DEV,)),
            pltpu.SemaphoreType.DMA((N_DEQ,)),
        ],
        compiler_params=pltpu.CompilerParams(
            collective_id=0, vmem_limit_bytes=56 * 1024 * 1024
        ),
    )(x, w_mat)
    return out.reshape(N_ROW, N_COL)
